# baseline (speedup 1.0000x reference)
import numpy as np
import jax
import jax.numpy as jnp
from jax.experimental import pallas as pl
from jax.experimental.pallas import tpu as pltpu

_BB = 2048

_D1 = np.zeros((5, 28, 28), np.float32)
_D2 = np.zeros((5, 14, 14), np.float32)
for _j in range(5):
    for _w in range(28):
        _v = _w + _j - 2
        if 0 <= _v < 28:
            _D1[_j, _v, _w] = 1.0
for _j in range(5):
    for _w in range(14):
        _v = _w + _j - 2
        if 0 <= _v < 14:
            _D2[_j, _v, _w] = 1.0


def _pack_weights(w1, b1, w2, b2, wf1, bf1):
    bf16 = jnp.bfloat16
    d1 = jnp.asarray(_D1)
    d2 = jnp.asarray(_D2)
    w1r = w1[:, :10].reshape(5, 5, 10)
    t1 = jnp.einsum('jvw,ijc->ivwc', d1, w1r)
    t1 = t1.reshape(5, 28, 14, 2, 10).transpose(0, 1, 3, 2, 4)
    t1 = jnp.pad(t1.reshape(5, 28, 2, 140), ((0, 0), (0, 0), (0, 0), (0, 116)))
    t1 = jnp.pad(t1.reshape(5, 28, 512), ((0, 0), (0, 4), (0, 0)))
    b1p = jnp.pad(jnp.tile(b1[0, :10], 14), (0, 116))
    b1p = jnp.tile(b1p, 2)
    T1 = t1.at[2, 28].set(b1p).reshape(160, 512)
    w2r = w2[:, :10, :20].reshape(5, 5, 10, 20)
    t2 = jnp.einsum('jvw,ijcd->ivcwd', d2, w2r)
    t2 = t2.reshape(5, 140, 7, 2, 20).transpose(0, 1, 3, 2, 4)
    t2 = jnp.pad(t2.reshape(5, 140, 2, 140), ((0, 0), (0, 0), (0, 0), (0, 116)))
    T2 = jnp.pad(t2.reshape(5, 140, 512), ((0, 0), (0, 116), (0, 0))).reshape(1280, 512)
    b2p = jnp.pad(jnp.tile(b2[0, :20], 7), (0, 116))
    b2p = jnp.tile(b2p, 2).reshape(1, 512)
    F1 = jnp.pad(wf1.reshape(7, 140, 128), ((0, 0), (0, 116), (0, 0)))
    return T1.astype(bf16), T2.astype(bf16), b2p, F1.astype(bf16)


def _lenet_kernel(x_ref, t1_ref, t2_ref, b2_ref, f1_ref, bf1_ref,
                  wf2_ref, bf2_ref, o_ref, a1_s):
    f32 = jnp.float32
    bf16 = jnp.bfloat16
    xb = x_ref[...]
    for h2 in range(14):
        pooled_w = []
        for hp in range(2):
            h = 2 * h2 + hp
            i0 = max(0, 2 - h)
            i1 = min(4, 29 - h)
            lhs = xb[:, 32 * (h - 2 + i0): 32 * (h - 2 + i1) + 32]
            rhs = t1_ref[32 * i0: 32 * (i1 + 1), :]
            acc = jnp.dot(lhs, rhs, preferred_element_type=f32)
            y = jnp.maximum(acc, 0.0).astype(bf16)
            pooled_w.append(jnp.maximum(y[:, :256], y[:, 256:]))
        a1_s[:, 256 * h2: 256 * h2 + 256] = jnp.maximum(pooled_w[0],
                                                        pooled_w[1])

    accf = None
    for h2o in range(7):
        pooled_w = []
        for hp in range(2):
            h = 2 * h2o + hp
            i0 = max(0, 2 - h)
            i1 = min(4, 15 - h)
            lhs = a1_s[:, 256 * (h - 2 + i0): 256 * (h - 2 + i1) + 256]
            rhs = t2_ref[256 * i0: 256 * (i1 + 1), :]
            acc = jnp.dot(lhs, rhs, preferred_element_type=f32)
            y = jnp.maximum(acc + b2_ref[...], 0.0).astype(bf16)
            pooled_w.append(jnp.maximum(y[:, :256], y[:, 256:]))
        feat = jnp.maximum(pooled_w[0], pooled_w[1])
        d = jnp.dot(feat, f1_ref[h2o], preferred_element_type=f32)
        accf = d if accf is None else accf + d

    hid = (accf + bf1_ref[...]).astype(jnp.bfloat16)
    out = jnp.dot(hid, wf2_ref[...], preferred_element_type=f32)
    o_ref[...] = out + bf2_ref[...]


def kernel(w1, b1, w2, b2, wf1, bf1, wf2, bf2, x_nchw):
    B = x_nchw.shape[0]
    bf16 = jnp.bfloat16
    T1, T2, b2p, F1 = _pack_weights(w1, b1, w2, b2, wf1, bf1)
    xb = jnp.pad(x_nchw.reshape(B, 28, 28),
                 ((0, 0), (0, 0), (0, 4)),
                 constant_values=1.0).reshape(B, 896).astype(bf16)

    out = pl.pallas_call(
        _lenet_kernel,
        out_shape=jax.ShapeDtypeStruct((B, 128), jnp.float32),
        grid=(B // _BB,),
        in_specs=[
            pl.BlockSpec((_BB, 896), lambda i: (i, 0)),
            pl.BlockSpec((160, 512), lambda i: (0, 0)),
            pl.BlockSpec((1280, 512), lambda i: (0, 0)),
            pl.BlockSpec((1, 512), lambda i: (0, 0)),
            pl.BlockSpec((7, 256, 128), lambda i: (0, 0, 0)),
            pl.BlockSpec((1, 128), lambda i: (0, 0)),
            pl.BlockSpec((128, 128), lambda i: (0, 0)),
            pl.BlockSpec((1, 128), lambda i: (0, 0)),
        ],
        out_specs=pl.BlockSpec((_BB, 128), lambda i: (i, 0)),
        scratch_shapes=[pltpu.VMEM((_BB, 3584), bf16)],
        compiler_params=pltpu.CompilerParams(
            dimension_semantics=("parallel",)),
    )(xb, T1, T2, b2p, F1, bf1, wf2.astype(bf16), bf2)
    return out[:, :10]


# speedup vs baseline: 1.2105x; 1.2105x over previous
import numpy as np
import jax
import jax.numpy as jnp
from jax.experimental import pallas as pl
from jax.experimental.pallas import tpu as pltpu

_BB = 1024

_D1 = np.zeros((5, 28, 28), np.float32)
_D2 = np.zeros((5, 14, 14), np.float32)
for _j in range(5):
    for _w in range(28):
        _v = _w + _j - 2
        if 0 <= _v < 28:
            _D1[_j, _v, _w] = 1.0
for _j in range(5):
    for _w in range(14):
        _v = _w + _j - 2
        if 0 <= _v < 14:
            _D2[_j, _v, _w] = 1.0


def _pack_weights(w1, b1, w2, b2, wf1, bf1):
    bf16 = jnp.bfloat16
    d1 = jnp.asarray(_D1)
    d2 = jnp.asarray(_D2)
    w1r = w1[:, :10].reshape(5, 5, 10)
    t1 = jnp.einsum('jvw,ijc->ivwc', d1, w1r)
    t1 = t1.reshape(5, 28, 14, 2, 10).transpose(0, 1, 3, 2, 4)
    t1 = jnp.pad(t1.reshape(5, 28, 2, 140), ((0, 0), (0, 0), (0, 0), (0, 116)))
    t1 = jnp.pad(t1.reshape(5, 28, 512), ((0, 0), (0, 4), (0, 0)))
    b1p = jnp.pad(jnp.tile(b1[0, :10], 14), (0, 116))
    b1p = jnp.tile(b1p, 2)
    T1 = t1.at[2, 28].set(b1p).reshape(160, 512)
    w2r = w2[:, :10, :20].reshape(5, 5, 10, 20)
    t2 = jnp.einsum('jvw,ijcd->ivcwd', d2, w2r)
    t2 = t2.reshape(5, 140, 7, 2, 20).transpose(0, 1, 3, 2, 4)
    t2 = jnp.pad(t2.reshape(5, 140, 2, 140), ((0, 0), (0, 0), (0, 0), (0, 116)))
    T2 = jnp.pad(t2.reshape(5, 140, 512), ((0, 0), (0, 116), (0, 0))).reshape(1280, 512)
    b2p = jnp.pad(jnp.tile(b2[0, :20], 7), (0, 116))
    b2p = jnp.tile(b2p, 2).reshape(1, 512)
    F1 = jnp.pad(wf1.reshape(7, 140, 128), ((0, 0), (0, 116), (0, 0)))
    return T1.astype(bf16), T2.astype(bf16), b2p, F1.astype(bf16)


def _lenet_kernel(x_ref, t1_ref, t2_ref, b2_ref, f1_ref, bf1_ref,
                  wf2_ref, bf2_ref, o_ref, a1_s):
    f32 = jnp.float32
    bf16 = jnp.bfloat16
    xb = x_ref[...]
    for h2 in range(14):
        pooled_w = []
        for hp in range(2):
            h = 2 * h2 + hp
            i0 = max(0, 2 - h)
            i1 = min(4, 29 - h)
            lhs = xb[:, 32 * (h - 2 + i0): 32 * (h - 2 + i1) + 32]
            rhs = t1_ref[32 * i0: 32 * (i1 + 1), :]
            acc = jnp.dot(lhs, rhs, preferred_element_type=f32)
            y = jnp.maximum(acc, 0.0).astype(bf16)
            pooled_w.append(jnp.maximum(y[:, :256], y[:, 256:]))
        a1_s[:, 256 * h2: 256 * h2 + 256] = jnp.maximum(pooled_w[0],
                                                        pooled_w[1])

    accf = None
    for h2o in range(7):
        pooled_w = []
        for hp in range(2):
            h = 2 * h2o + hp
            i0 = max(0, 2 - h)
            i1 = min(4, 15 - h)
            lhs = a1_s[:, 256 * (h - 2 + i0): 256 * (h - 2 + i1) + 256]
            rhs = t2_ref[256 * i0: 256 * (i1 + 1), :]
            acc = jnp.dot(lhs, rhs, preferred_element_type=f32)
            y = jnp.maximum(acc + b2_ref[...], 0.0).astype(bf16)
            pooled_w.append(jnp.maximum(y[:, :256], y[:, 256:]))
        feat = jnp.maximum(pooled_w[0], pooled_w[1])
        d = jnp.dot(feat, f1_ref[h2o], preferred_element_type=f32)
        accf = d if accf is None else accf + d

    hid = (accf + bf1_ref[...]).astype(jnp.bfloat16)
    out = jnp.dot(hid, wf2_ref[...], preferred_element_type=f32)
    o_ref[...] = out + bf2_ref[...]


def kernel(w1, b1, w2, b2, wf1, bf1, wf2, bf2, x_nchw):
    B = x_nchw.shape[0]
    bf16 = jnp.bfloat16
    T1, T2, b2p, F1 = _pack_weights(w1, b1, w2, b2, wf1, bf1)
    xb = jnp.pad(x_nchw.reshape(B, 28, 28),
                 ((0, 0), (0, 0), (0, 4)),
                 constant_values=1.0).reshape(B, 896).astype(bf16)

    out = pl.pallas_call(
        _lenet_kernel,
        out_shape=jax.ShapeDtypeStruct((B, 128), jnp.float32),
        grid=(B // _BB,),
        in_specs=[
            pl.BlockSpec((_BB, 896), lambda i: (i, 0)),
            pl.BlockSpec((160, 512), lambda i: (0, 0)),
            pl.BlockSpec((1280, 512), lambda i: (0, 0)),
            pl.BlockSpec((1, 512), lambda i: (0, 0)),
            pl.BlockSpec((7, 256, 128), lambda i: (0, 0, 0)),
            pl.BlockSpec((1, 128), lambda i: (0, 0)),
            pl.BlockSpec((128, 128), lambda i: (0, 0)),
            pl.BlockSpec((1, 128), lambda i: (0, 0)),
        ],
        out_specs=pl.BlockSpec((_BB, 128), lambda i: (i, 0)),
        scratch_shapes=[pltpu.VMEM((_BB, 3584), bf16)],
        compiler_params=pltpu.CompilerParams(
            dimension_semantics=("parallel",)),
    )(xb, T1, T2, b2p, F1, bf1, wf2.astype(bf16), bf2)
    return out[:, :10]
